# baseline (speedup 1.0000x reference)
import functools

import jax
import jax.numpy as jnp
import numpy as np
from jax import lax
from jax.experimental import pallas as pl
from jax.experimental.pallas import tpu as pltpu
from jax.experimental.pallas import tpu_sc as plsc

B, N_E, D_R = 32, 128, 64
N_TRI = N_E * (N_E + 1) // 2
IDX_ROWS = 68
CHUNK = 512
N_FULL = N_TRI // CHUNK
TAIL = N_TRI - N_FULL * CHUNK


def _tri_indices() -> np.ndarray:
    rows, cols = np.tril_indices(N_E, k=0)
    idx = (rows * N_E + cols).astype(np.int32)
    pad = np.zeros(IDX_ROWS * 128, dtype=np.int32)
    pad[: idx.size] = idx
    return pad.reshape(IDX_ROWS, 128)


_IDX2D = _tri_indices()


def _flatten_tri_sc(x, idx):
    mesh = plsc.VectorSubcoreMesh(core_axis_name="c", subcore_axis_name="s")

    @functools.partial(
        pl.kernel,
        mesh=mesh,
        compiler_params=pltpu.CompilerParams(use_tc_tiling_on_sc=False),
        out_type=jax.ShapeDtypeStruct((B, N_TRI, D_R), jnp.float32),
        scratch_types=[
            pltpu.VMEM((IDX_ROWS, 128), jnp.int32),
            pltpu.VMEM((CHUNK, D_R), jnp.float32),
            pltpu.SemaphoreType.DMA,
        ],
    )
    def k(in_hbm, idx_hbm, out_hbm, idx_v, buf, sem):
        wid = lax.axis_index("s") * 2 + lax.axis_index("c")
        pltpu.sync_copy(idx_hbm, idx_v)

        def body(j, carry):
            copies = []
            for t in range(4):
                copies.append(
                    pltpu.async_copy(
                        in_hbm.at[wid].at[idx_v.at[j * 4 + t]],
                        buf.at[pl.ds(t * 128, 128)],
                        sem,
                    )
                )
            for c in copies:
                c.wait()
            pltpu.sync_copy(buf, out_hbm.at[wid, pl.ds(j * CHUNK, CHUNK)])
            return carry

        lax.fori_loop(0, N_FULL, body, 0)

        pltpu.async_copy(
            in_hbm.at[wid].at[idx_v.at[N_FULL * 4]],
            buf.at[pl.ds(0, 128)],
            sem,
        ).wait()
        pltpu.sync_copy(
            buf.at[pl.ds(0, TAIL)],
            out_hbm.at[wid, pl.ds(N_FULL * CHUNK, TAIL)],
        )

    return k(x, idx)


def kernel(inputs):
    b, n_e, _, d_r = inputs.shape
    table = inputs.reshape(b, n_e * n_e, d_r)
    idx = jnp.asarray(_IDX2D)
    out = _flatten_tri_sc(table, idx)
    return out.reshape(b, N_TRI * d_r)
